# baseline (speedup 1.0000x reference)
import functools

import jax
import jax.numpy as jnp
from jax import lax
from jax.experimental import pallas as pl
from jax.experimental.pallas import tpu as pltpu
from jax.experimental.pallas import tpu_sc as plsc

NC = 2
NS = 16
NW = NC * NS
CHUNK = 128


def _make_lookup(T, B0, V, D):
    B = T * B0
    DG, DR, BL = D // 8, 8, 128
    NBC = B0 // BL
    b_per_w = B // NW
    n_chunks = b_per_w // CHUNK
    NBUF = 4
    OBUF = 2

    mesh = plsc.VectorSubcoreMesh(
        core_axis_name="c", subcore_axis_name="s", num_cores=NC, num_subcores=NS
    )

    @functools.partial(
        pl.kernel,
        mesh=mesh,
        out_type=jax.ShapeDtypeStruct((T, DG, NBC, DR, BL), jnp.float32),
        scratch_types=[
            pltpu.VMEM((b_per_w,), jnp.int32),
            [pltpu.VMEM((CHUNK, D), jnp.float32) for _ in range(NBUF)],
            [pltpu.VMEM((1, DG, 1, DR, BL), jnp.float32) for _ in range(OBUF)],
            [pltpu.SemaphoreType.DMA for _ in range(NBUF)],
            [pltpu.SemaphoreType.DMA for _ in range(OBUF)],
        ],
        compiler_params=pltpu.CompilerParams(
            use_tc_tiling_on_sc=False, needs_layout_passes=False
        ),
    )
    def lookup(ids_hbm, table_hbm, out_hbm, idx_v, rows, obufs, gsems, wsems):
        wid = lax.axis_index("s") * NC + lax.axis_index("c")
        base = wid * b_per_w
        c0 = wid * n_chunks
        pltpu.sync_copy(ids_hbm.at[pl.ds(base, b_per_w)], idx_v)

        def gather(j, b):
            pltpu.async_copy(
                table_hbm.at[idx_v.at[pl.ds(j * CHUNK, CHUNK)]], rows[b], gsems[b]
            )

        def wait_gather(j, b):
            pltpu.make_async_copy(
                table_hbm.at[idx_v.at[pl.ds(j * CHUNK, CHUNK)]], rows[b], gsems[b]
            ).wait()

        def out_slice(j):
            c = c0 + j
            t = c // NBC
            bc = lax.rem(c, NBC)
            return out_hbm.at[pl.ds(t, 1), :, pl.ds(bc, 1)]

        def wb(j, ob):
            pltpu.async_copy(obufs[ob], out_slice(j), wsems[ob])

        def wait_wb(j, ob):
            pltpu.make_async_copy(obufs[ob], out_slice(j), wsems[ob]).wait()

        iota16 = lax.iota(jnp.int32, 16)
        ridxs = [iota16 + (g * 16) for g in range(BL // 16)]
        dvecs = [jnp.full((16,), d, dtype=jnp.int32) for d in range(D)]

        def transpose(b, ob):
            src = rows[b]
            dst = obufs[ob]
            for dg in range(DG):
                for dr in range(DR):
                    d = dg * DR + dr
                    for g in range(BL // 16):
                        dst[0, dg, 0, dr, pl.ds(g * 16, 16)] = plsc.load_gather(
                            src, [ridxs[g], dvecs[d]]
                        )

        for j0 in range(3):
            gather(j0, j0)

        def body(i, carry):
            for u in range(4):
                j = i * 4 + u
                b = u
                ob = u % OBUF

                @pl.when(j >= OBUF)
                def _():
                    wait_wb(j - OBUF, ob)

                wait_gather(j, b)
                transpose(b, ob)
                wb(j, ob)

                @pl.when(j + 3 < n_chunks)
                def _():
                    gather(j + 3, (u + 3) % NBUF)

            return carry

        lax.fori_loop(0, n_chunks // 4, body, 0)

        wait_wb(n_chunks - 2, (n_chunks - 2) % OBUF)
        wait_wb(n_chunks - 1, (n_chunks - 1) % OBUF)

    return lookup


def kernel(enum_ids, table):
    B0, T = enum_ids.shape
    V, D = table.shape
    ids = enum_ids.T.reshape(T * B0).astype(jnp.int32)
    out5 = _make_lookup(T, B0, V, D)(ids, table)
    return out5.transpose(2, 4, 0, 1, 3).reshape(B0, T, D)


# speedup vs baseline: 1.2663x; 1.2663x over previous
import functools

import jax
import jax.numpy as jnp
from jax import lax
from jax.experimental import pallas as pl
from jax.experimental.pallas import tpu as pltpu
from jax.experimental.pallas import tpu_sc as plsc

NC = 2
NS = 16
NW = NC * NS
CHUNK = 128


def _make_lookup(T, B0, V, D):
    B = T * B0
    DG, DR, BL = D // 8, 8, 128
    NBC = B0 // BL
    b_per_w = B // NW
    n_chunks = b_per_w // CHUNK
    NBUF = 4
    OBUF = 2

    mesh = plsc.VectorSubcoreMesh(
        core_axis_name="c", subcore_axis_name="s", num_cores=NC, num_subcores=NS
    )

    @functools.partial(
        pl.kernel,
        mesh=mesh,
        out_type=jax.ShapeDtypeStruct((T, DG, NBC, DR, BL), jnp.float32),
        scratch_types=[
            pltpu.VMEM((b_per_w,), jnp.int32),
            [pltpu.VMEM((CHUNK, D), jnp.float32) for _ in range(NBUF)],
            [pltpu.VMEM((1, DG, 1, DR, BL), jnp.float32) for _ in range(OBUF)],
            [pltpu.SemaphoreType.DMA for _ in range(NBUF)],
            [pltpu.SemaphoreType.DMA for _ in range(OBUF)],
        ],
        compiler_params=pltpu.CompilerParams(
            use_tc_tiling_on_sc=False, needs_layout_passes=False
        ),
    )
    def lookup(ids_hbm, table_hbm, out_hbm, idx_v, rows, obufs, gsems, wsems):
        wid = lax.axis_index("s") * NC + lax.axis_index("c")
        base = wid * b_per_w
        c0 = wid * n_chunks
        pltpu.sync_copy(ids_hbm.at[pl.ds(base, b_per_w)], idx_v)

        def gather(j, b):
            pltpu.async_copy(
                table_hbm.at[idx_v.at[pl.ds(j * CHUNK, CHUNK)]], rows[b], gsems[b]
            )

        def wait_gather(j, b):
            pltpu.make_async_copy(
                table_hbm.at[idx_v.at[pl.ds(j * CHUNK, CHUNK)]], rows[b], gsems[b]
            ).wait()

        def out_slice(j):
            c = c0 + j
            t = c // NBC
            bc = lax.rem(c, NBC)
            return out_hbm.at[pl.ds(t, 1), :, pl.ds(bc, 1)]

        def wb(j, ob):
            pltpu.async_copy(obufs[ob], out_slice(j), wsems[ob])

        def wait_wb(j, ob):
            pltpu.make_async_copy(obufs[ob], out_slice(j), wsems[ob]).wait()

        iota16 = lax.iota(jnp.int32, 16)
        zero16 = jnp.zeros((16,), dtype=jnp.int32)

        def transpose(b, ob):
            src = rows[b]
            dst = obufs[ob]

            @plsc.parallel_loop(0, D * (BL // 16), 1, unroll=8)
            def _(i):
                d = i // 8
                g = i % 8
                ridx = iota16 + g * 16
                v = plsc.load_gather(src, [ridx, zero16 + d])
                plsc.store_scatter(
                    dst,
                    [zero16, zero16 + d // DR, zero16, zero16 + d % DR, ridx],
                    v,
                )

        for j0 in range(3):
            gather(j0, j0)

        def body(i, carry):
            for u in range(4):
                j = i * 4 + u
                b = u
                ob = u % OBUF

                @pl.when(j >= OBUF)
                def _():
                    wait_wb(j - OBUF, ob)

                wait_gather(j, b)
                transpose(b, ob)
                wb(j, ob)

                @pl.when(j + 3 < n_chunks)
                def _():
                    gather(j + 3, (u + 3) % NBUF)

            return carry

        lax.fori_loop(0, n_chunks // 4, body, 0)

        wait_wb(n_chunks - 2, (n_chunks - 2) % OBUF)
        wait_wb(n_chunks - 1, (n_chunks - 1) % OBUF)

    return lookup


def kernel(enum_ids, table):
    B0, T = enum_ids.shape
    V, D = table.shape
    ids = enum_ids.T.reshape(T * B0).astype(jnp.int32)
    out5 = _make_lookup(T, B0, V, D)(ids, table)
    return out5.transpose(2, 4, 0, 1, 3).reshape(B0, T, D)


# speedup vs baseline: 1.2798x; 1.0107x over previous
import functools

import jax
import jax.numpy as jnp
from jax import lax
from jax.experimental import pallas as pl
from jax.experimental.pallas import tpu as pltpu
from jax.experimental.pallas import tpu_sc as plsc

NC = 2
NS = 16
NW = NC * NS
CHUNK = 128


def _make_lookup(T, B0, V, D):
    B = T * B0
    DG, DR, BL = D // 8, 8, 128
    NBC = B0 // BL
    b_per_w = B // NW
    n_chunks = b_per_w // CHUNK
    NBUF = 4
    OBUF = 2

    mesh = plsc.VectorSubcoreMesh(
        core_axis_name="c", subcore_axis_name="s", num_cores=NC, num_subcores=NS
    )

    @functools.partial(
        pl.kernel,
        mesh=mesh,
        out_type=jax.ShapeDtypeStruct((T * DG * NBC * DR * BL,), jnp.float32),
        scratch_types=[
            pltpu.VMEM((b_per_w,), jnp.int32),
            [pltpu.VMEM((CHUNK, D), jnp.float32) for _ in range(NBUF)],
            [pltpu.VMEM((DG * DR * BL,), jnp.float32) for _ in range(OBUF)],
            [pltpu.SemaphoreType.DMA for _ in range(NBUF)],
            [pltpu.SemaphoreType.DMA for _ in range(OBUF)],
        ],
        compiler_params=pltpu.CompilerParams(
            use_tc_tiling_on_sc=False, needs_layout_passes=False
        ),
    )
    def lookup(ids_hbm, table_hbm, out_hbm, idx_v, rows, obufs, gsems, wsems):
        wid = lax.axis_index("s") * NC + lax.axis_index("c")
        base = wid * b_per_w
        c0 = wid * n_chunks
        pltpu.sync_copy(ids_hbm.at[pl.ds(base, b_per_w)], idx_v)

        def gather(j, b):
            pltpu.async_copy(
                table_hbm.at[idx_v.at[pl.ds(j * CHUNK, CHUNK)]], rows[b], gsems[b]
            )

        def wait_gather(j, b):
            pltpu.make_async_copy(
                table_hbm.at[idx_v.at[pl.ds(j * CHUNK, CHUNK)]], rows[b], gsems[b]
            ).wait()

        TB = DR * BL

        def _piece(j, dg):
            c = c0 + j
            t = c // NBC
            bc = lax.rem(c, NBC)
            return (t * (DG * NBC) + dg * NBC + bc) * TB

        def wb(j, ob):
            for dg in range(DG):
                pltpu.async_copy(
                    obufs[ob].at[pl.ds(dg * TB, TB)],
                    out_hbm.at[pl.ds(_piece(j, dg), TB)],
                    wsems[ob],
                )

        def wait_wb(j, ob):
            for dg in range(DG):
                pltpu.make_async_copy(
                    obufs[ob].at[pl.ds(dg * TB, TB)],
                    out_hbm.at[pl.ds(_piece(j, dg), TB)],
                    wsems[ob],
                ).wait()

        iota16 = lax.iota(jnp.int32, 16)
        zero16 = jnp.zeros((16,), dtype=jnp.int32)

        def transpose(b, ob):
            src = rows[b]
            dst = obufs[ob]

            @plsc.parallel_loop(0, D * (BL // 16), 1, unroll=16)
            def _(i):
                d = i // 8
                g = i % 8
                v = plsc.load_gather(src, [iota16 + g * 16, zero16 + d])
                dbase = (d // DR) * TB + lax.rem(d, DR) * BL + g * 16
                plsc.store_scatter(dst, [iota16 + dbase], v)

        for j0 in range(3):
            gather(j0, j0)

        def body(i, carry):
            for u in range(4):
                j = i * 4 + u
                b = u
                ob = u % OBUF

                @pl.when(j >= OBUF)
                def _():
                    wait_wb(j - OBUF, ob)

                wait_gather(j, b)
                transpose(b, ob)
                wb(j, ob)

                @pl.when(j + 3 < n_chunks)
                def _():
                    gather(j + 3, (u + 3) % NBUF)

            return carry

        lax.fori_loop(0, n_chunks // 4, body, 0)

        wait_wb(n_chunks - 2, (n_chunks - 2) % OBUF)
        wait_wb(n_chunks - 1, (n_chunks - 1) % OBUF)

    return lookup


def kernel(enum_ids, table):
    B0, T = enum_ids.shape
    V, D = table.shape
    ids = enum_ids.T.reshape(T * B0).astype(jnp.int32)
    out1 = _make_lookup(T, B0, V, D)(ids, table)
    out5 = out1.reshape(T, D // 8, B0 // 128, 8, 128)
    return out5.transpose(2, 4, 0, 1, 3).reshape(B0, T, D)


# speedup vs baseline: 1.3521x; 1.0565x over previous
import functools

import jax
import jax.numpy as jnp
from jax import lax
from jax.experimental import pallas as pl
from jax.experimental.pallas import tpu as pltpu
from jax.experimental.pallas import tpu_sc as plsc

NC = 2
NS = 16
NW = NC * NS
CHUNK = 128


def _make_lookup(T, B0, V, D):
    B = T * B0
    DG, DR, BL = D // 8, 8, 128
    NBC = B0 // BL
    b_per_w = B // NW
    n_chunks = b_per_w // CHUNK
    NBUF = 4
    OBUF = 2

    mesh = plsc.VectorSubcoreMesh(
        core_axis_name="c", subcore_axis_name="s", num_cores=NC, num_subcores=NS
    )

    @functools.partial(
        pl.kernel,
        mesh=mesh,
        out_type=jax.ShapeDtypeStruct((T * DG * NBC * DR * BL,), jnp.float32),
        scratch_types=[
            pltpu.VMEM((b_per_w,), jnp.int32),
            pltpu.VMEM((b_per_w,), jnp.int32),
            [pltpu.VMEM((CHUNK, 4 * D), jnp.float32) for _ in range(NBUF)],
            [pltpu.VMEM((DG * DR * BL,), jnp.float32) for _ in range(OBUF)],
            [pltpu.SemaphoreType.DMA for _ in range(NBUF)],
            [pltpu.SemaphoreType.DMA for _ in range(OBUF)],
        ],
        compiler_params=pltpu.CompilerParams(
            use_tc_tiling_on_sc=True, needs_layout_passes=False
        ),
    )
    def lookup(ids_hbm, table_hbm, out_hbm, idx_v, idx2_v, rows, obufs, gsems, wsems):
        wid = lax.axis_index("s") * NC + lax.axis_index("c")
        base = wid * b_per_w
        c0 = wid * n_chunks
        pltpu.sync_copy(ids_hbm.at[pl.ds(base, b_per_w)], idx_v)

        @plsc.parallel_loop(0, b_per_w // 16, 1, unroll=8)
        def _(i):
            w = idx_v[pl.ds(i * 16, 16)]
            idx2_v[pl.ds(i * 16, 16)] = lax.shift_right_logical(w, 2)
            idx_v[pl.ds(i * 16, 16)] = (w & 3) * D

        def gather(j, b):
            pltpu.async_copy(
                table_hbm.at[idx2_v.at[pl.ds(j * CHUNK, CHUNK)]], rows[b], gsems[b]
            )

        def wait_gather(j, b):
            pltpu.make_async_copy(
                table_hbm.at[idx2_v.at[pl.ds(j * CHUNK, CHUNK)]], rows[b], gsems[b]
            ).wait()

        TB = DR * BL

        def _piece(j, dg):
            c = c0 + j
            t = c // NBC
            bc = lax.rem(c, NBC)
            return (t * (DG * NBC) + dg * NBC + bc) * TB

        def wb(j, ob):
            for dg in range(DG):
                pltpu.async_copy(
                    obufs[ob].at[pl.ds(dg * TB, TB)],
                    out_hbm.at[pl.ds(_piece(j, dg), TB)],
                    wsems[ob],
                )

        def wait_wb(j, ob):
            for dg in range(DG):
                pltpu.make_async_copy(
                    obufs[ob].at[pl.ds(dg * TB, TB)],
                    out_hbm.at[pl.ds(_piece(j, dg), TB)],
                    wsems[ob],
                ).wait()

        iota16 = lax.iota(jnp.int32, 16)
        zero16 = jnp.zeros((16,), dtype=jnp.int32)

        def transpose(j, b, ob):
            src = rows[b]
            dst = obufs[ob]
            j0 = j * CHUNK

            @plsc.parallel_loop(0, D * (BL // 16), 1, unroll=16)
            def _(i):
                d = i // 8
                g = i % 8
                woff = idx_v[pl.ds(j0 + g * 16, 16)]
                v = plsc.load_gather(src, [iota16 + g * 16, woff + d])
                dbase = (d // DR) * TB + lax.rem(d, DR) * BL + g * 16
                plsc.store_scatter(dst, [iota16 + dbase], v)

        for j0 in range(3):
            gather(j0, j0)

        def body(i, carry):
            for u in range(4):
                j = i * 4 + u
                b = u
                ob = u % OBUF

                @pl.when(j >= OBUF)
                def _():
                    wait_wb(j - OBUF, ob)

                wait_gather(j, b)
                transpose(j, b, ob)
                wb(j, ob)

                @pl.when(j + 3 < n_chunks)
                def _():
                    gather(j + 3, (u + 3) % NBUF)

            return carry

        lax.fori_loop(0, n_chunks // 4, body, 0)

        wait_wb(n_chunks - 2, (n_chunks - 2) % OBUF)
        wait_wb(n_chunks - 1, (n_chunks - 1) % OBUF)

    return lookup


def kernel(enum_ids, table):
    B0, T = enum_ids.shape
    V, D = table.shape
    ids = enum_ids.T.reshape(T * B0).astype(jnp.int32)
    table4 = table.reshape(V // 4, 4 * D)
    out1 = _make_lookup(T, B0, V, D)(ids, table4)
    out5 = out1.reshape(T, D // 8, B0 // 128, 8, 128)
    return out5.transpose(2, 4, 0, 1, 3).reshape(B0, T, D)


# speedup vs baseline: 1.3798x; 1.0205x over previous
import functools

import jax
import jax.numpy as jnp
from jax import lax
from jax.experimental import pallas as pl
from jax.experimental.pallas import tpu as pltpu
from jax.experimental.pallas import tpu_sc as plsc

NC = 2
NS = 16
NW = NC * NS
CHUNK = 128


def _make_lookup(T, B0, V, D):
    B = T * B0
    DG, DR, BL = D // 8, 8, 128
    NBC = B0 // BL
    b_per_w = B // NW
    n_chunks = b_per_w // CHUNK
    NBUF = 4
    OBUF = 2

    mesh = plsc.VectorSubcoreMesh(
        core_axis_name="c", subcore_axis_name="s", num_cores=NC, num_subcores=NS
    )

    @functools.partial(
        pl.kernel,
        mesh=mesh,
        out_type=jax.ShapeDtypeStruct((T * DG * NBC * DR * BL,), jnp.float32),
        scratch_types=[
            pltpu.VMEM((b_per_w,), jnp.int32),
            pltpu.VMEM((b_per_w,), jnp.int32),
            [pltpu.VMEM((CHUNK, 4 * D), jnp.float32) for _ in range(NBUF)],
            [pltpu.VMEM((DG * DR * BL,), jnp.float32) for _ in range(OBUF)],
            [pltpu.SemaphoreType.DMA for _ in range(NBUF)],
            [pltpu.SemaphoreType.DMA for _ in range(OBUF)],
        ],
        compiler_params=pltpu.CompilerParams(
            use_tc_tiling_on_sc=True, needs_layout_passes=False
        ),
    )
    def lookup(ids_hbm, table_hbm, out_hbm, idx_v, idx2_v, rows, obufs, gsems, wsems):
        wid = lax.axis_index("s") * NC + lax.axis_index("c")
        base = wid * b_per_w
        c0 = wid * n_chunks
        pltpu.sync_copy(ids_hbm.at[pl.ds(base, b_per_w)], idx_v)

        @plsc.parallel_loop(0, b_per_w // 16, 1, unroll=8)
        def _(i):
            w = idx_v[pl.ds(i * 16, 16)]
            idx2_v[pl.ds(i * 16, 16)] = lax.shift_right_logical(w, 2)
            idx_v[pl.ds(i * 16, 16)] = (w & 3) * D

        def gather(j, b):
            pltpu.async_copy(
                table_hbm.at[idx2_v.at[pl.ds(j * CHUNK, CHUNK)]], rows[b], gsems[b]
            )

        def wait_gather(j, b):
            pltpu.make_async_copy(
                table_hbm.at[idx2_v.at[pl.ds(j * CHUNK, CHUNK)]], rows[b], gsems[b]
            ).wait()

        TB = DR * BL

        def _piece(j, dg):
            c = c0 + j
            t = c // NBC
            bc = lax.rem(c, NBC)
            return (t * (DG * NBC) + dg * NBC + bc) * TB

        def wb(j, ob):
            for dg in range(DG):
                pltpu.async_copy(
                    obufs[ob].at[pl.ds(dg * TB, TB)],
                    out_hbm.at[pl.ds(_piece(j, dg), TB)],
                    wsems[ob],
                )

        def wait_wb(j, ob):
            for dg in range(DG):
                pltpu.make_async_copy(
                    obufs[ob].at[pl.ds(dg * TB, TB)],
                    out_hbm.at[pl.ds(_piece(j, dg), TB)],
                    wsems[ob],
                ).wait()

        iota16 = lax.iota(jnp.int32, 16)
        zero16 = jnp.zeros((16,), dtype=jnp.int32)

        def transpose(j, b, ob):
            src = rows[b]
            dst = obufs[ob]
            j0 = j * CHUNK

            @plsc.parallel_loop(0, D * (BL // 16), 1, unroll=32)
            def _(i):
                d = i // 8
                g = i % 8
                woff = idx_v[pl.ds(j0 + g * 16, 16)]
                v = plsc.load_gather(src, [iota16 + g * 16, woff + d])
                dbase = (d // DR) * TB + lax.rem(d, DR) * BL + g * 16
                plsc.store_scatter(dst, [iota16 + dbase], v)

        for j0 in range(3):
            gather(j0, j0)

        def body(i, carry):
            for u in range(4):
                j = i * 4 + u
                b = u
                ob = u % OBUF

                @pl.when(j >= OBUF)
                def _():
                    wait_wb(j - OBUF, ob)

                wait_gather(j, b)
                transpose(j, b, ob)
                wb(j, ob)

                @pl.when(j + 3 < n_chunks)
                def _():
                    gather(j + 3, (u + 3) % NBUF)

            return carry

        lax.fori_loop(0, n_chunks // 4, body, 0)

        wait_wb(n_chunks - 2, (n_chunks - 2) % OBUF)
        wait_wb(n_chunks - 1, (n_chunks - 1) % OBUF)

    return lookup


def kernel(enum_ids, table):
    B0, T = enum_ids.shape
    V, D = table.shape
    ids = enum_ids.T.reshape(T * B0).astype(jnp.int32)
    table4 = table.reshape(V // 4, 4 * D)
    out1 = _make_lookup(T, B0, V, D)(ids, table4)
    out5 = out1.reshape(T, D // 8, B0 // 128, 8, 128)
    return out5.transpose(2, 4, 0, 1, 3).reshape(B0, T, D)
